# Initial kernel scaffold; baseline (speedup 1.0000x reference)
import jax
import jax.numpy as jnp
from jax.experimental import pallas as pl


def kernel(x, Wr, W1, b1, W2, b2):
    raise NotImplementedError("write your pallas kernel here")


# speedup vs baseline: 1.1072x; 1.1072x over previous
import functools

import jax
import jax.numpy as jnp
from jax.experimental import pallas as pl
from jax.experimental.pallas import tpu as pltpu

B, S, D = 1, 2048, 768
FF = 3072
E = 8
TOPK = 2

TBLK = 256
FFBLK = 512
NT = S // TBLK
NFF = FF // FFBLK
LANES = 128

NEG = -1e30


def _moe_dense_kernel(x_ref, wr_ref, w1_ref, b1_ref, w2_ref, b2_ref,
                      out_ref, idx_ref, w_ref, sel_ref, acc_ref):
    e = pl.program_id(1)
    blk = pl.program_id(2)

    lane = jax.lax.broadcasted_iota(jnp.int32, (TBLK, LANES), 1)

    @pl.when(jnp.logical_and(e == 0, blk == 0))
    def _router():
        logits_full = jnp.dot(x_ref[...], wr_ref[...],
                              preferred_element_type=jnp.float32)
        valid = lane < E
        logits = jnp.where(valid, logits_full, NEG)
        m1 = jnp.max(logits, axis=1, keepdims=True)
        i1 = jnp.min(jnp.where(logits == m1, lane, LANES), axis=1,
                     keepdims=True)
        logits2 = jnp.where(lane == i1, NEG, logits)
        m2 = jnp.max(logits2, axis=1, keepdims=True)
        i2 = jnp.min(jnp.where(logits2 == m2, lane, LANES), axis=1,
                     keepdims=True)
        e2 = jnp.exp(m2 - m1)
        z = 1.0 + e2
        w1 = 1.0 / z
        w2 = e2 / z
        sel_ref[...] = (jnp.where(lane == i1, w1, 0.0)
                        + jnp.where(lane == i2, w2, 0.0))
        idx_ref[...] = jnp.where(lane == 0, i1,
                                 jnp.where(lane == 1, i2, 0))
        w_ref[...] = jnp.where(lane == 0, w1,
                               jnp.where(lane == 1, w2, 0.0))

    gate = jnp.sum(jnp.where(lane == e, sel_ref[...], 0.0), axis=1,
                   keepdims=True)

    @pl.when(jnp.logical_and(e == 0, blk == 0))
    def _zero_acc():
        acc_ref[...] = jnp.zeros_like(acc_ref)

    @pl.when(blk == 0)
    def _bias2():
        acc_ref[...] += gate * b2_ref[0, 0, :][None, :]

    h = jnp.dot(x_ref[...], w1_ref[0], preferred_element_type=jnp.float32)
    h = h + b1_ref[0, 0, :][None, :]
    h = 0.5 * h * (1.0 + jax.lax.erf(h * 0.7071067811865476))
    acc_ref[...] += jnp.dot(gate * h, w2_ref[0],
                            preferred_element_type=jnp.float32)

    @pl.when(jnp.logical_and(e == E - 1, blk == NFF - 1))
    def _emit():
        out_ref[...] = acc_ref[...]


@jax.jit
def kernel(x, Wr, W1, b1, W2, b2):
    x_flat = x.reshape(-1, D)
    wr_pad = jnp.zeros((D, LANES), jnp.float32).at[:, :E].set(Wr)
    b1_r = b1.reshape(E * NFF, 1, FFBLK)
    b2_r = b2.reshape(E, 1, D)

    grid = (NT, E, NFF)
    out, idx_pad, w_pad = pl.pallas_call(
        _moe_dense_kernel,
        grid=grid,
        in_specs=[
            pl.BlockSpec((TBLK, D), lambda t, e, b: (t, 0)),
            pl.BlockSpec((D, LANES), lambda t, e, b: (0, 0)),
            pl.BlockSpec((1, D, FFBLK), lambda t, e, b: (e, 0, b)),
            pl.BlockSpec((1, 1, FFBLK), lambda t, e, b: (e * NFF + b, 0, 0)),
            pl.BlockSpec((1, FFBLK, D), lambda t, e, b: (e, b, 0)),
            pl.BlockSpec((1, 1, D), lambda t, e, b: (e, 0, 0)),
        ],
        out_specs=[
            pl.BlockSpec((TBLK, D), lambda t, e, b: (t, 0)),
            pl.BlockSpec((TBLK, LANES), lambda t, e, b: (t, 0)),
            pl.BlockSpec((TBLK, LANES), lambda t, e, b: (t, 0)),
        ],
        out_shape=[
            jax.ShapeDtypeStruct((S, D), jnp.float32),
            jax.ShapeDtypeStruct((S, LANES), jnp.int32),
            jax.ShapeDtypeStruct((S, LANES), jnp.float32),
        ],
        scratch_shapes=[
            pltpu.VMEM((TBLK, LANES), jnp.float32),
            pltpu.VMEM((TBLK, D), jnp.float32),
        ],
        compiler_params=pltpu.CompilerParams(
            dimension_semantics=("arbitrary", "arbitrary", "arbitrary"),
        ),
    )(x_flat, wr_pad, W1, b1_r, W2, b2_r)

    output = out.reshape(B, S, D)
    top_idx = idx_pad[:, :TOPK]
    expert_weights = w_pad[:, :TOPK]
    return output, top_idx, expert_weights
